# baseline (speedup 1.0000x reference)
import jax
import jax.numpy as jnp
from jax.experimental import pallas as pl
from jax.experimental.pallas import tpu as pltpu

_BLOCK_N = 2048


def _logits_kernel(x_ref, w_ref, b_ref, o_ref):
    o_ref[...] = (
        jax.lax.dot_general(
            x_ref[...],
            w_ref[...],
            (((1,), (1,)), ((), ())),
            preferred_element_type=jnp.float32,
        )
        + b_ref[...]
    )


def kernel(inputs, labels, W, b):
    del labels
    B, D = inputs.shape
    N = W.shape[0]
    b2 = b.reshape(1, N)
    grid = (pl.cdiv(N, _BLOCK_N),)
    return pl.pallas_call(
        _logits_kernel,
        grid=grid,
        in_specs=[
            pl.BlockSpec((B, D), lambda i: (0, 0)),
            pl.BlockSpec((_BLOCK_N, D), lambda i: (i, 0)),
            pl.BlockSpec((1, _BLOCK_N), lambda i: (0, i)),
        ],
        out_specs=pl.BlockSpec((B, _BLOCK_N), lambda i: (0, i)),
        out_shape=jax.ShapeDtypeStruct((B, N), jnp.float32),
        compiler_params=pltpu.CompilerParams(
            dimension_semantics=("arbitrary",),
        ),
    )(inputs, W, b2)


# speedup vs baseline: 3.8275x; 3.8275x over previous
import jax
import jax.numpy as jnp
from jax.experimental import pallas as pl
from jax.experimental.pallas import tpu as pltpu

_BLOCK_N = 2048


def _logits_kernel(wa_ref, xa_ref, o_ref):
    o_ref[...] = jax.lax.dot_general(
        wa_ref[...],
        xa_ref[...],
        (((0,), (0,)), ((), ())),
        preferred_element_type=jnp.float32,
    )


def kernel(inputs, labels, W, b):
    del labels
    B, D = inputs.shape
    N = W.shape[0]
    K = D + 1
    wa = jnp.concatenate([W, b[:, None]], axis=1).T
    xa = jnp.concatenate(
        [inputs, jnp.ones((B, 1), dtype=inputs.dtype)], axis=1
    ).T
    grid = (pl.cdiv(N, _BLOCK_N),)
    out_t = pl.pallas_call(
        _logits_kernel,
        grid=grid,
        in_specs=[
            pl.BlockSpec((K, _BLOCK_N), lambda i: (0, i)),
            pl.BlockSpec((K, B), lambda i: (0, 0)),
        ],
        out_specs=pl.BlockSpec((_BLOCK_N, B), lambda i: (i, 0)),
        out_shape=jax.ShapeDtypeStruct((N, B), jnp.float32),
        compiler_params=pltpu.CompilerParams(
            dimension_semantics=("arbitrary",),
        ),
    )(wa, xa)
    return out_t.T


# speedup vs baseline: 4.1692x; 1.0893x over previous
import jax
import jax.numpy as jnp
from jax.experimental import pallas as pl
from jax.experimental.pallas import tpu as pltpu

_BLOCK_N = 2048


def _logits_kernel(wt_ref, xt_ref, b_ref, o_ref):
    wk = jnp.concatenate([wt_ref[...], b_ref[...][None, :]], axis=0)
    xk = jnp.concatenate(
        [xt_ref[...], jnp.ones((1, xt_ref.shape[1]), dtype=jnp.float32)], axis=0
    )
    o_ref[...] = jax.lax.dot_general(
        wk,
        xk,
        (((0,), (0,)), ((), ())),
        preferred_element_type=jnp.float32,
    )


def kernel(inputs, labels, W, b):
    del labels
    B, D = inputs.shape
    N = W.shape[0]
    wt = W.T
    xt = inputs.T
    grid = (pl.cdiv(N, _BLOCK_N),)
    out_t = pl.pallas_call(
        _logits_kernel,
        grid=grid,
        in_specs=[
            pl.BlockSpec((D, _BLOCK_N), lambda i: (0, i)),
            pl.BlockSpec((D, B), lambda i: (0, 0)),
            pl.BlockSpec((_BLOCK_N,), lambda i: (i,)),
        ],
        out_specs=pl.BlockSpec((_BLOCK_N, B), lambda i: (i, 0)),
        out_shape=jax.ShapeDtypeStruct((N, B), jnp.float32),
        compiler_params=pltpu.CompilerParams(
            dimension_semantics=("arbitrary",),
        ),
    )(wt, xt, b)
    return out_t.T


# speedup vs baseline: 4.1905x; 1.0051x over previous
import jax
import jax.numpy as jnp
from jax.experimental import pallas as pl
from jax.experimental.pallas import tpu as pltpu

_BLOCK_N = 4096


def _logits_kernel(wt_ref, xt_ref, b_ref, o_ref):
    wk = jnp.concatenate([wt_ref[...], b_ref[...][None, :]], axis=0)
    xk = jnp.concatenate(
        [xt_ref[...], jnp.ones((1, xt_ref.shape[1]), dtype=jnp.float32)], axis=0
    )
    o_ref[...] = jax.lax.dot_general(
        wk,
        xk,
        (((0,), (0,)), ((), ())),
        preferred_element_type=jnp.float32,
    )


def kernel(inputs, labels, W, b):
    del labels
    B, D = inputs.shape
    N = W.shape[0]
    wt = W.T
    xt = inputs.T
    grid = (pl.cdiv(N, _BLOCK_N),)
    out_t = pl.pallas_call(
        _logits_kernel,
        grid=grid,
        in_specs=[
            pl.BlockSpec((D, _BLOCK_N), lambda i: (0, i)),
            pl.BlockSpec((D, B), lambda i: (0, 0)),
            pl.BlockSpec((_BLOCK_N,), lambda i: (i,)),
        ],
        out_specs=pl.BlockSpec((_BLOCK_N, B), lambda i: (i, 0)),
        out_shape=jax.ShapeDtypeStruct((N, B), jnp.float32),
        compiler_params=pltpu.CompilerParams(
            dimension_semantics=("arbitrary",),
        ),
    )(wt, xt, b)
    return out_t.T
